# baseline (speedup 1.0000x reference)
import functools

import jax
import jax.numpy as jnp
from jax import lax
from jax.experimental import pallas as pl
from jax.experimental.pallas import tpu as pltpu
from jax.experimental.pallas import tpu_sc as plsc

N_NODES = 10000
N_EDGES = 320000
D_FEAT = 128
NHID = 128
D_OUT = 20
D_OUT_PAD = 32

N_ROWS = 10112
ROWS_PT = N_ROWS // 16

NW = 32
CHUNK = 80
E_TOT = N_EDGES + N_NODES
NCH = 132
EPW = NCH * CHUNK
E_PAD = EPW * NW
NBUF = 4
NIB = NBUF


def _tc1_body(x_ref, w_ref, asv_ref, adv_ref, xp_ref, as_ref, ad_ref):
    xp = jnp.dot(x_ref[...], w_ref[...], preferred_element_type=jnp.float32)
    xp_ref[...] = xp
    as_ref[...] = jnp.sum(xp * asv_ref[...], axis=1, keepdims=True)
    ad_ref[...] = jnp.sum(xp * adv_ref[...], axis=1, keepdims=True)


def _tc_project(x, w, att_s, att_d):
    n, dout = x.shape[0], w.shape[1]
    return pl.pallas_call(
        _tc1_body,
        out_shape=[
            jax.ShapeDtypeStruct((n, dout), jnp.float32),
            jax.ShapeDtypeStruct((n, 1), jnp.float32),
            jax.ShapeDtypeStruct((n, 1), jnp.float32),
        ],
    )(x, w, att_s.reshape(1, -1), att_d.reshape(1, -1))


def _tc2_body(acc_ref, den_ref, b_ref, w_ref, asv_ref, adv_ref,
              xp_ref, as_ref, ad_ref):
    num = acc_ref[0] + acc_ref[1]
    den = den_ref[0] + den_ref[1] + 1e-16
    h = jnp.maximum(num / den + b_ref[...], 0.0)
    xp = jnp.dot(h, w_ref[...], preferred_element_type=jnp.float32)
    xp_ref[...] = xp
    as_ref[...] = jnp.sum(xp * asv_ref[...], axis=1, keepdims=True)
    ad_ref[...] = jnp.sum(xp * adv_ref[...], axis=1, keepdims=True)


def _tc_mid(acc, den, b, w, att_s, att_d):
    n, dout = acc.shape[1], w.shape[1]
    return pl.pallas_call(
        _tc2_body,
        out_shape=[
            jax.ShapeDtypeStruct((n, dout), jnp.float32),
            jax.ShapeDtypeStruct((n, 1), jnp.float32),
            jax.ShapeDtypeStruct((n, 1), jnp.float32),
        ],
    )(acc, den.reshape(2, n, 1), b.reshape(1, -1),
      w, att_s.reshape(1, -1), att_d.reshape(1, -1))


def _tc3_body(acc_ref, den_ref, b_ref, out_ref):
    num = acc_ref[0] + acc_ref[1]
    den = den_ref[0] + den_ref[1] + 1e-16
    out_ref[...] = num / den + b_ref[...]


def _tc_final(acc, den, b):
    n, dout = acc.shape[1], acc.shape[2]
    return pl.pallas_call(
        _tc3_body,
        out_shape=jax.ShapeDtypeStruct((n, dout), jnp.float32),
    )(acc, den.reshape(2, n, 1), b.reshape(1, -1))


@functools.lru_cache(maxsize=None)
def _make_sc_layer(d):
    mesh = plsc.VectorSubcoreMesh(core_axis_name="c", subcore_axis_name="s")

    @functools.partial(
        pl.kernel,
        mesh=mesh,
        compiler_params=pltpu.CompilerParams(use_tc_tiling_on_sc=False),
        out_type=[
            jax.ShapeDtypeStruct((2, N_ROWS, d), jnp.float32),
            jax.ShapeDtypeStruct((2, N_ROWS), jnp.float32),
        ],
        scratch_types=[
            pltpu.VMEM((NIB, 2, CHUNK), jnp.int32),
            pltpu.VMEM((NBUF, CHUNK), jnp.float32),
            pltpu.VMEM((NBUF, CHUNK), jnp.float32),
            pltpu.VMEM((NBUF, CHUNK, d), jnp.float32),
            pltpu.VMEM_SHARED((N_ROWS, d), jnp.float32),
            pltpu.VMEM_SHARED((N_ROWS,), jnp.float32),
        ] + [pltpu.SemaphoreType.DMA] * (2 * NBUF),
    )
    def sc_layer(xp_hbm, edge_hbm, asrc_hbm, adst_hbm,
                 zrow_hbm, zden_hbm,
                 out_hbm, dout_hbm,
                 eidx, av, wv, rows, acc, den,
                 *sems):
        semg = sems[:NBUF]
        semsc = sems[NBUF:2 * NBUF]
        c = lax.axis_index("c")
        s = lax.axis_index("s")
        wid = s * 2 + c
        r0 = s * ROWS_PT

        pltpu.sync_copy(zrow_hbm.at[pl.ds(r0, ROWS_PT)],
                        acc.at[pl.ds(r0, ROWS_PT)])
        pltpu.sync_copy(zden_hbm.at[pl.ds(r0, ROWS_PT)],
                        den.at[pl.ds(r0, ROWS_PT)])
        plsc.subcore_barrier()

        def load_idx(q, j):
            pltpu.sync_copy(edge_hbm.at[wid, j], eidx.at[q])

        def fire_gather(b, q):
            pltpu.async_copy(asrc_hbm.at[eidx.at[q, 0]], av.at[b], semg[b])
            pltpu.async_copy(adst_hbm.at[eidx.at[q, 1]], wv.at[b], semg[b])
            pltpu.async_copy(xp_hbm.at[eidx.at[q, 0]], rows.at[b], semg[b])

        def drain_gather(b, q):
            pltpu.make_async_copy(asrc_hbm.at[eidx.at[q, 0]], av.at[b],
                                  semg[b]).wait()
            pltpu.make_async_copy(adst_hbm.at[eidx.at[q, 1]], wv.at[b],
                                  semg[b]).wait()
            pltpu.make_async_copy(xp_hbm.at[eidx.at[q, 0]], rows.at[b],
                                  semg[b]).wait()

        def fire_scatter(b, q):
            pltpu.async_copy(wv.at[b], den.at[eidx.at[q, 1]], semsc[b],
                             add=True)
            pltpu.async_copy(rows.at[b], acc.at[eidx.at[q, 1]], semsc[b],
                             add=True)

        def drain_scatter(b, q):
            pltpu.make_async_copy(wv.at[b], den.at[eidx.at[q, 1]],
                                  semsc[b]).wait()
            pltpu.make_async_copy(rows.at[b], acc.at[eidx.at[q, 1]],
                                  semsc[b]).wait()

        def compute(b):
            for i in range(CHUNK // 16):
                sl = pl.ds(i * 16, 16)
                e = av[b, sl] + wv[b, sl]
                e = jnp.where(e >= 0.0, e, e * 0.2)
                wv[b, sl] = jnp.exp(e)

            def scale16(j2, c2):
                i0 = j2 * 16
                wvec = wv[b, pl.ds(i0, 16)]
                for k in range(16):
                    wsc = wvec[k]
                    for g in range(d // 16):
                        sl = pl.ds(g * 16, 16)
                        rows[b, i0 + k, sl] = rows[b, i0 + k, sl] * wsc
                return c2

            lax.fori_loop(0, CHUNK // 16, scale16, 0)

        load_idx(0, 0)
        load_idx(1, 1)
        fire_gather(0, 0)
        fire_gather(1, 1)

        nsteps = NCH // NBUF

        def outer(t, carry):
            for b in range(NBUF):
                j = NBUF * t + b
                drain_gather(b, b)
                compute(b)
                fire_scatter(b, b)
                nb = (b + 2) % NBUF

                def _next(nb=nb, j=j):
                    drain_scatter(nb, nb)
                    load_idx(nb, j + 2)
                    fire_gather(nb, nb)

                if b <= 1:
                    pl.when(t > 0)(lambda: drain_scatter(nb, nb))
                    load_idx(nb, j + 2)
                    fire_gather(nb, nb)
                else:
                    pl.when(t < nsteps - 1)(_next)
                    pl.when(t == nsteps - 1)(lambda: drain_scatter(nb, nb))
            return carry

        lax.fori_loop(0, nsteps, outer, 0)
        drain_scatter((NCH - 2) % NBUF, (NCH - 2) % NBUF)
        drain_scatter((NCH - 1) % NBUF, (NCH - 1) % NBUF)
        plsc.subcore_barrier()
        pltpu.sync_copy(acc.at[pl.ds(r0, ROWS_PT)],
                        out_hbm.at[c, pl.ds(r0, ROWS_PT)])
        pltpu.sync_copy(den.at[pl.ds(r0, ROWS_PT)],
                        dout_hbm.at[c, pl.ds(r0, ROWS_PT)])

    return sc_layer


def kernel(x, edge_index, W1, att_src1, att_dst1, b1,
           W2, att_src2, att_dst2, b2):
    loop = jnp.arange(N_NODES, dtype=jnp.int32)
    pad = N_NODES + jnp.arange(E_PAD - E_TOT, dtype=jnp.int32) % (
        N_ROWS - N_NODES)
    src = jnp.concatenate([edge_index[0].astype(jnp.int32), loop, pad])
    dst = jnp.concatenate([edge_index[1].astype(jnp.int32), loop, pad])
    edges = jnp.stack([src.reshape(NW, NCH, CHUNK),
                       dst.reshape(NW, NCH, CHUNK)], axis=2)

    xpad = jnp.pad(x, ((0, N_ROWS - N_NODES), (0, 0)))
    zrow = jnp.zeros((N_ROWS, NHID), jnp.float32)
    zden = jnp.zeros((N_ROWS,), jnp.float32)

    xp1, a_s1, a_d1 = _tc_project(xpad, W1, att_src1, att_dst1)
    acc1, den1 = _make_sc_layer(NHID)(xp1, edges,
                            a_s1.reshape(-1), a_d1.reshape(-1),
                            zrow, zden)

    W2p = jnp.pad(W2, ((0, 0), (0, D_OUT_PAD - D_OUT)))
    as2p = jnp.pad(att_src2, (0, D_OUT_PAD - D_OUT))
    ad2p = jnp.pad(att_dst2, (0, D_OUT_PAD - D_OUT))
    b1p = b1
    xp2, a_s2, a_d2 = _tc_mid(acc1, den1, b1p, W2p, as2p, ad2p)

    acc2, den2 = _make_sc_layer(D_OUT_PAD)(xp2, edges,
                            a_s2.reshape(-1), a_d2.reshape(-1),
                            zrow[:, :D_OUT_PAD], zden)

    b2p = jnp.pad(b2, (0, D_OUT_PAD - D_OUT))
    out = _tc_final(acc2, den2, b2p)
    return out[:N_NODES, :D_OUT]


# speedup vs baseline: 1.1127x; 1.1127x over previous
import functools

import jax
import jax.numpy as jnp
from jax import lax
from jax.experimental import pallas as pl
from jax.experimental.pallas import tpu as pltpu
from jax.experimental.pallas import tpu_sc as plsc

N_NODES = 10000
N_EDGES = 320000
D_FEAT = 128
NHID = 128
D_OUT = 20
D_OUT_PAD = 32

N_ROWS = 10112
ROWS_PT = N_ROWS // 16

NW = 32
CHUNK = 112
E_TOT = N_EDGES + N_NODES
NCH = 93
EPW = NCH * CHUNK
E_PAD = EPW * NW
NBUF = 3
NIB = NBUF


def _tc1_body(x_ref, w_ref, asv_ref, adv_ref, xp_ref, as_ref, ad_ref):
    xp = jnp.dot(x_ref[...], w_ref[...], preferred_element_type=jnp.float32)
    xp_ref[...] = xp
    as_ref[...] = jnp.sum(xp * asv_ref[...], axis=1, keepdims=True)
    ad_ref[...] = jnp.sum(xp * adv_ref[...], axis=1, keepdims=True)


def _tc_project(x, w, att_s, att_d):
    n, dout = x.shape[0], w.shape[1]
    return pl.pallas_call(
        _tc1_body,
        out_shape=[
            jax.ShapeDtypeStruct((n, dout), jnp.float32),
            jax.ShapeDtypeStruct((n, 1), jnp.float32),
            jax.ShapeDtypeStruct((n, 1), jnp.float32),
        ],
    )(x, w, att_s.reshape(1, -1), att_d.reshape(1, -1))


def _tc2_body(acc_ref, den_ref, b_ref, w_ref, asv_ref, adv_ref,
              xp_ref, as_ref, ad_ref):
    num = acc_ref[0] + acc_ref[1]
    den = den_ref[0] + den_ref[1] + 1e-16
    h = jnp.maximum(num / den + b_ref[...], 0.0)
    xp = jnp.dot(h, w_ref[...], preferred_element_type=jnp.float32)
    xp_ref[...] = xp
    as_ref[...] = jnp.sum(xp * asv_ref[...], axis=1, keepdims=True)
    ad_ref[...] = jnp.sum(xp * adv_ref[...], axis=1, keepdims=True)


def _tc_mid(acc, den, b, w, att_s, att_d):
    n, dout = acc.shape[1], w.shape[1]
    return pl.pallas_call(
        _tc2_body,
        out_shape=[
            jax.ShapeDtypeStruct((n, dout), jnp.float32),
            jax.ShapeDtypeStruct((n, 1), jnp.float32),
            jax.ShapeDtypeStruct((n, 1), jnp.float32),
        ],
    )(acc, den.reshape(2, n, 1), b.reshape(1, -1),
      w, att_s.reshape(1, -1), att_d.reshape(1, -1))


def _tc3_body(acc_ref, den_ref, b_ref, out_ref):
    num = acc_ref[0] + acc_ref[1]
    den = den_ref[0] + den_ref[1] + 1e-16
    out_ref[...] = num / den + b_ref[...]


def _tc_final(acc, den, b):
    n, dout = acc.shape[1], acc.shape[2]
    return pl.pallas_call(
        _tc3_body,
        out_shape=jax.ShapeDtypeStruct((n, dout), jnp.float32),
    )(acc, den.reshape(2, n, 1), b.reshape(1, -1))


@functools.lru_cache(maxsize=None)
def _make_sc_layer(d):
    mesh = plsc.VectorSubcoreMesh(core_axis_name="c", subcore_axis_name="s")

    @functools.partial(
        pl.kernel,
        mesh=mesh,
        compiler_params=pltpu.CompilerParams(use_tc_tiling_on_sc=False),
        out_type=[
            jax.ShapeDtypeStruct((2, N_ROWS, d), jnp.float32),
            jax.ShapeDtypeStruct((2, N_ROWS), jnp.float32),
        ],
        scratch_types=[
            pltpu.VMEM((NIB, 2, CHUNK), jnp.int32),
            pltpu.VMEM((NBUF, CHUNK), jnp.float32),
            pltpu.VMEM((NBUF, CHUNK), jnp.float32),
            pltpu.VMEM((NBUF, CHUNK, d), jnp.float32),
            pltpu.VMEM_SHARED((N_ROWS, d), jnp.float32),
            pltpu.VMEM_SHARED((N_ROWS,), jnp.float32),
        ] + [pltpu.SemaphoreType.DMA] * (2 * NBUF),
    )
    def sc_layer(xp_hbm, edge_hbm, asrc_hbm, adst_hbm,
                 zrow_hbm, zden_hbm,
                 out_hbm, dout_hbm,
                 eidx, av, wv, rows, acc, den,
                 *sems):
        semg = sems[:NBUF]
        semsc = sems[NBUF:2 * NBUF]
        c = lax.axis_index("c")
        s = lax.axis_index("s")
        wid = s * 2 + c
        r0 = s * ROWS_PT

        pltpu.sync_copy(zrow_hbm.at[pl.ds(r0, ROWS_PT)],
                        acc.at[pl.ds(r0, ROWS_PT)])
        pltpu.sync_copy(zden_hbm.at[pl.ds(r0, ROWS_PT)],
                        den.at[pl.ds(r0, ROWS_PT)])
        plsc.subcore_barrier()

        def load_idx(q, j):
            pltpu.sync_copy(edge_hbm.at[wid, j], eidx.at[q])

        def fire_gather(b, q):
            pltpu.async_copy(asrc_hbm.at[eidx.at[q, 0]], av.at[b], semg[b])
            pltpu.async_copy(adst_hbm.at[eidx.at[q, 1]], wv.at[b], semg[b])
            pltpu.async_copy(xp_hbm.at[eidx.at[q, 0]], rows.at[b], semg[b])

        def drain_gather(b, q):
            pltpu.make_async_copy(asrc_hbm.at[eidx.at[q, 0]], av.at[b],
                                  semg[b]).wait()
            pltpu.make_async_copy(adst_hbm.at[eidx.at[q, 1]], wv.at[b],
                                  semg[b]).wait()
            pltpu.make_async_copy(xp_hbm.at[eidx.at[q, 0]], rows.at[b],
                                  semg[b]).wait()

        def fire_scatter(b, q):
            pltpu.async_copy(wv.at[b], den.at[eidx.at[q, 1]], semsc[b],
                             add=True)
            pltpu.async_copy(rows.at[b], acc.at[eidx.at[q, 1]], semsc[b],
                             add=True)

        def drain_scatter(b, q):
            pltpu.make_async_copy(wv.at[b], den.at[eidx.at[q, 1]],
                                  semsc[b]).wait()
            pltpu.make_async_copy(rows.at[b], acc.at[eidx.at[q, 1]],
                                  semsc[b]).wait()

        def compute(b):
            for i in range(CHUNK // 16):
                sl = pl.ds(i * 16, 16)
                e = av[b, sl] + wv[b, sl]
                e = jnp.where(e >= 0.0, e, e * 0.2)
                wv[b, sl] = jnp.exp(e)

            def scale16(j2, c2):
                i0 = j2 * 16
                wvec = wv[b, pl.ds(i0, 16)]
                for k in range(16):
                    wsc = wvec[k]
                    for g in range(d // 16):
                        sl = pl.ds(g * 16, 16)
                        rows[b, i0 + k, sl] = rows[b, i0 + k, sl] * wsc
                return c2

            lax.fori_loop(0, CHUNK // 16, scale16, 0)

        load_idx(0, 0)
        load_idx(1, 1)
        fire_gather(0, 0)
        fire_gather(1, 1)

        nsteps = NCH // NBUF

        def outer(t, carry):
            for b in range(NBUF):
                j = NBUF * t + b
                drain_gather(b, b)
                compute(b)
                fire_scatter(b, b)
                nb = (b + 2) % NBUF

                def _next(nb=nb, j=j):
                    drain_scatter(nb, nb)
                    load_idx(nb, j + 2)
                    fire_gather(nb, nb)

                if b == 0:
                    pl.when(t > 0)(lambda: drain_scatter(nb, nb))
                    load_idx(nb, j + 2)
                    fire_gather(nb, nb)
                else:
                    pl.when(t < nsteps - 1)(_next)
                    pl.when(t == nsteps - 1)(lambda: drain_scatter(nb, nb))
            return carry

        lax.fori_loop(0, nsteps, outer, 0)
        drain_scatter((NCH - 1) % NBUF, (NCH - 1) % NBUF)
        plsc.subcore_barrier()
        pltpu.sync_copy(acc.at[pl.ds(r0, ROWS_PT)],
                        out_hbm.at[c, pl.ds(r0, ROWS_PT)])
        pltpu.sync_copy(den.at[pl.ds(r0, ROWS_PT)],
                        dout_hbm.at[c, pl.ds(r0, ROWS_PT)])

    return sc_layer


def kernel(x, edge_index, W1, att_src1, att_dst1, b1,
           W2, att_src2, att_dst2, b2):
    loop = jnp.arange(N_NODES, dtype=jnp.int32)
    pad = N_NODES + jnp.arange(E_PAD - E_TOT, dtype=jnp.int32) % (
        N_ROWS - N_NODES)
    src = jnp.concatenate([edge_index[0].astype(jnp.int32), loop, pad])
    dst = jnp.concatenate([edge_index[1].astype(jnp.int32), loop, pad])
    edges = jnp.stack([src.reshape(NW, NCH, CHUNK),
                       dst.reshape(NW, NCH, CHUNK)], axis=2)

    xpad = jnp.pad(x, ((0, N_ROWS - N_NODES), (0, 0)))
    zrow = jnp.zeros((N_ROWS, NHID), jnp.float32)
    zden = jnp.zeros((N_ROWS,), jnp.float32)

    xp1, a_s1, a_d1 = _tc_project(xpad, W1, att_src1, att_dst1)
    acc1, den1 = _make_sc_layer(NHID)(xp1, edges,
                            a_s1.reshape(-1), a_d1.reshape(-1),
                            zrow, zden)

    W2p = jnp.pad(W2, ((0, 0), (0, D_OUT_PAD - D_OUT)))
    as2p = jnp.pad(att_src2, (0, D_OUT_PAD - D_OUT))
    ad2p = jnp.pad(att_dst2, (0, D_OUT_PAD - D_OUT))
    b1p = b1
    xp2, a_s2, a_d2 = _tc_mid(acc1, den1, b1p, W2p, as2p, ad2p)

    acc2, den2 = _make_sc_layer(D_OUT_PAD)(xp2, edges,
                            a_s2.reshape(-1), a_d2.reshape(-1),
                            zrow[:, :D_OUT_PAD], zden)

    b2p = jnp.pad(b2, (0, D_OUT_PAD - D_OUT))
    out = _tc_final(acc2, den2, b2p)
    return out[:N_NODES, :D_OUT]
